# baseline (speedup 1.0000x reference)
import functools

import jax
import jax.numpy as jnp
from jax import lax
from jax.experimental import pallas as pl
from jax.experimental.pallas import tpu as pltpu
from jax.experimental.pallas import tpu_sc as plsc

_VOCAB = 1000000
_DIM = 32
_B = 4096
_L = 200
_NTOK = _B * _L

_info = plsc.get_sparse_core_info()
_NC = _info.num_cores
_NS = _info.num_subcores
_NW = _NC * _NS
_B_PER_W = _NTOK // _NW

_NBUF = 4
_CHUNK = 800
_N_CHUNKS = _B_PER_W // _CHUNK
_N_GROUPS = _N_CHUNKS // _NBUF

_mesh = plsc.VectorSubcoreMesh(core_axis_name="c", subcore_axis_name="s")


@functools.partial(
    pl.kernel,
    mesh=_mesh,
    out_type=jax.ShapeDtypeStruct((_NTOK, _DIM), jnp.float32),
    scratch_types=[
        pltpu.VMEM((_NBUF, _CHUNK), jnp.int32),
        pltpu.VMEM((_NBUF, _CHUNK, _DIM), jnp.float32),
    ] + [pltpu.SemaphoreType.DMA] * (3 * _NBUF),
    compiler_params=pltpu.CompilerParams(use_tc_tiling_on_sc=False),
)
def _gather_kernel(idx_hbm, table_hbm, out_hbm, idx_v, rows_v, *sems):
    sem_idx = sems[0:_NBUF]
    sem_g = sems[_NBUF:2 * _NBUF]
    sem_st = sems[2 * _NBUF:3 * _NBUF]

    wid = lax.axis_index("s") * _NC + lax.axis_index("c")
    wbase = pl.multiple_of(wid * _B_PER_W, 8)

    def idx_copy(i, b):
        base = pl.multiple_of(wbase + i * _CHUNK, 8)
        return pltpu.make_async_copy(
            idx_hbm.at[pl.ds(base, _CHUNK)], idx_v.at[b], sem_idx[b])

    def gather_copy(b):
        return pltpu.make_async_copy(
            table_hbm.at[idx_v.at[b]], rows_v.at[b], sem_g[b])

    def store_copy(i, b):
        base = pl.multiple_of(wbase + i * _CHUNK, 8)
        return pltpu.make_async_copy(
            rows_v.at[b], out_hbm.at[pl.ds(base, _CHUNK)], sem_st[b])

    for b in range(_NBUF):
        idx_copy(jnp.int32(b), b).start()

    def retire(ip, bp):
        gather_copy(bp).wait()
        store_copy(ip, bp).start()

        @pl.when(ip + _NBUF < _N_CHUNKS)
        def _():
            idx_copy(ip + _NBUF, bp).start()

    def group(g, carry):
        for b in range(_NBUF):
            i = g * _NBUF + b
            idx_copy(i, b).wait()

            @pl.when(g >= 1)
            def _():
                store_copy(i - _NBUF, b).wait()

            gather_copy(b).start()

            bp = (b - 1) % _NBUF
            if b == 0:
                @pl.when(g >= 1)
                def _():
                    retire(i - 1, bp)
            else:
                retire(i - 1, bp)
        return carry

    lax.fori_loop(0, _N_GROUPS, group, 0)

    last = _N_CHUNKS - 1
    retire(last, (_N_CHUNKS - 1) % _NBUF)
    for b in range(_NBUF):
        store_copy(_N_CHUNKS - _NBUF + b, b).wait()


def kernel(inputs, table):
    idx = inputs.reshape(-1)
    out = _gather_kernel(idx, table)
    return out.reshape(_B, _L, _DIM)


# speedup vs baseline: 1.0013x; 1.0013x over previous
import functools

import jax
import jax.numpy as jnp
from jax import lax
from jax.experimental import pallas as pl
from jax.experimental.pallas import tpu as pltpu
from jax.experimental.pallas import tpu_sc as plsc

_VOCAB = 1000000
_DIM = 32
_B = 4096
_L = 200
_NTOK = _B * _L

_info = plsc.get_sparse_core_info()
_NC = _info.num_cores
_NS = _info.num_subcores
_NW = _NC * _NS
_B_PER_W = _NTOK // _NW

_NBUF = 2
_CHUNK = 1600
_N_CHUNKS = _B_PER_W // _CHUNK
_N_GROUPS = _N_CHUNKS // _NBUF

_mesh = plsc.VectorSubcoreMesh(core_axis_name="c", subcore_axis_name="s")


@functools.partial(
    pl.kernel,
    mesh=_mesh,
    out_type=jax.ShapeDtypeStruct((_NTOK, _DIM), jnp.float32),
    scratch_types=[
        pltpu.VMEM((_NBUF, _CHUNK), jnp.int32),
        pltpu.VMEM((_NBUF, _CHUNK, _DIM), jnp.float32),
    ] + [pltpu.SemaphoreType.DMA] * (3 * _NBUF),
    compiler_params=pltpu.CompilerParams(use_tc_tiling_on_sc=False),
)
def _gather_kernel(idx_hbm, table_hbm, out_hbm, idx_v, rows_v, *sems):
    sem_idx = sems[0:_NBUF]
    sem_g = sems[_NBUF:2 * _NBUF]
    sem_st = sems[2 * _NBUF:3 * _NBUF]

    wid = lax.axis_index("s") * _NC + lax.axis_index("c")
    wbase = pl.multiple_of(wid * _B_PER_W, 8)

    def idx_copy(i, b):
        base = pl.multiple_of(wbase + i * _CHUNK, 8)
        return pltpu.make_async_copy(
            idx_hbm.at[pl.ds(base, _CHUNK)], idx_v.at[b], sem_idx[b])

    def gather_copy(b):
        return pltpu.make_async_copy(
            table_hbm.at[idx_v.at[b]], rows_v.at[b], sem_g[b])

    def store_copy(i, b):
        base = pl.multiple_of(wbase + i * _CHUNK, 8)
        return pltpu.make_async_copy(
            rows_v.at[b], out_hbm.at[pl.ds(base, _CHUNK)], sem_st[b])

    for b in range(_NBUF):
        idx_copy(jnp.int32(b), b).start()

    def retire(ip, bp):
        gather_copy(bp).wait()
        store_copy(ip, bp).start()

        @pl.when(ip + _NBUF < _N_CHUNKS)
        def _():
            idx_copy(ip + _NBUF, bp).start()

    def group(g, carry):
        for b in range(_NBUF):
            i = g * _NBUF + b
            idx_copy(i, b).wait()

            @pl.when(g >= 1)
            def _():
                store_copy(i - _NBUF, b).wait()

            gather_copy(b).start()

            bp = (b - 1) % _NBUF
            if b == 0:
                @pl.when(g >= 1)
                def _():
                    retire(i - 1, bp)
            else:
                retire(i - 1, bp)
        return carry

    lax.fori_loop(0, _N_GROUPS, group, 0)

    last = _N_CHUNKS - 1
    retire(last, (_N_CHUNKS - 1) % _NBUF)
    for b in range(_NBUF):
        store_copy(_N_CHUNKS - _NBUF + b, b).wait()


def kernel(inputs, table):
    idx = inputs.reshape(-1)
    out = _gather_kernel(idx, table)
    return out.reshape(_B, _L, _DIM)
